# baseline (speedup 1.0000x reference)
import jax
import jax.numpy as jnp
from jax import lax
from jax.experimental import pallas as pl
from jax.experimental.pallas import tpu as pltpu
from jax.experimental.pallas import tpu_sc as plsc

PATHS = 16
N_TOKENS = 16384
D_MODEL = 2048

NC = 2
NS = 16
NW = NC * NS
B_PER_W = N_TOKENS // NW
CHUNK = 16
NCHUNK = B_PER_W // CHUNK
NBUF = 3


def _worker_id():
  return lax.axis_index("s") * NC + lax.axis_index("c")


def _route_body(score_hbm, route_hbm, hist_hbm, score_v, route_v, hist_v):
  wid = _worker_id()
  pltpu.sync_copy(score_hbm.at[pl.ds(wid * B_PER_W * PATHS, B_PER_W * PATHS)],
                  score_v)
  lane = lax.iota(jnp.int32, 16)

  @pl.loop(0, NCHUNK, init_carry=jnp.zeros(16, jnp.int32))
  def hist_loop(g, hist):
    flat = (g * CHUNK + lane) * PATHS
    best = plsc.load_gather(score_v, [flat])
    arg = jnp.zeros(16, jnp.int32)
    for p in range(1, PATHS):
      vals = plsc.load_gather(score_v, [flat + p])
      m = vals > best
      best = jnp.where(m, vals, best)
      arg = jnp.where(m, p, arg)
    route_v[g] = arg
    for p in range(PATHS):
      cnt = plsc.all_reduce_population_count(arg == p)
      hist = hist + jnp.where(lane == p, cnt, 0)
    return hist

  hist_v[...] = hist_loop
  pltpu.sync_copy(route_v, route_hbm.at[pl.ds(wid * NCHUNK, NCHUNK)])
  pltpu.sync_copy(hist_v, hist_hbm.at[wid])


def _dispatch_body(inputs_hbm, route_hbm, hist_hbm,
                   out_hbm, counts_hbm,
                   hist_all_v, route_v, carry_v, tot_v,
                   rows0, rows1, rows2, g0, g1, g2, s0, s1, s2):
  wid = _worker_id()
  row_base = wid * B_PER_W

  bufs = (rows0, rows1, rows2)
  gsems = (g0, g1, g2)
  ssems = (s0, s1, s2)

  def start_in(k, b):
    pltpu.async_copy(
        inputs_hbm.at[pl.ds(row_base + k * CHUNK, CHUNK)], bufs[b], gsems[b])

  for b in range(NBUF):
    start_in(b, b)

  pltpu.sync_copy(hist_hbm, hist_all_v)
  pltpu.sync_copy(route_hbm.at[pl.ds(wid * NCHUNK, NCHUNK)], route_v)
  lane = lax.iota(jnp.int32, 16)

  widv = jnp.full((16,), 0, jnp.int32) + wid
  total = jnp.zeros(16, jnp.int32)
  prior = jnp.zeros(16, jnp.int32)
  for w2 in range(NW):
    h = hist_all_v[w2]
    total = total + h
    prior = prior + jnp.where(jnp.full((16,), w2, jnp.int32) < widv, h, 0)
  path_base = plsc.cumsum(total) - total
  carry_v[...] = path_base + prior

  tot_v[...] = total

  @pl.when(wid == 0)
  def _():
    pltpu.sync_copy(tot_v, counts_hbm)

  def drain_store(b):
    pltpu.make_async_copy(
        inputs_hbm.at[pl.ds(row_base, CHUNK)], bufs[b], ssems[b]).wait()

  @pl.loop(0, NCHUNK, step=NBUF)
  def _(j):
    for b in range(NBUF):
      k = j + b

      @pl.when(k < NCHUNK)
      def _():
        arg = route_v[k]
        dest_base = plsc.load_gather(carry_v, [arg])
        rank = jnp.zeros(16, jnp.int32)
        ghist = jnp.zeros(16, jnp.int32)
        for p in range(PATHS):
          m = arg == p
          cs = plsc.cumsum(m.astype(jnp.int32))
          rank = jnp.where(m, cs - 1, rank)
          ghist = ghist + jnp.where(lane == p,
                                    plsc.all_reduce_population_count(m), 0)
        dest = dest_base + rank
        carry_v[...] = carry_v[...] + ghist
        pltpu.make_async_copy(
            inputs_hbm.at[pl.ds(row_base + k * CHUNK, CHUNK)], bufs[b],
            gsems[b]).wait()
        pltpu.async_copy(bufs[b], out_hbm.at[dest], ssems[b])

    for b in range(NBUF):
      k = j + b

      @pl.when(k + NBUF < NCHUNK)
      def _():
        drain_store(b)
        start_in(k + NBUF, b)

  for b in range(NBUF):
    drain_store(b)


def _mesh():
  return plsc.VectorSubcoreMesh(
      core_axis_name="c", subcore_axis_name="s", num_cores=NC,
      num_subcores=NS)


@jax.jit
def _run(inputs, score):
  route2d, hist = pl.kernel(
      _route_body,
      out_type=(jax.ShapeDtypeStruct((N_TOKENS // CHUNK, CHUNK), jnp.int32),
                jax.ShapeDtypeStruct((NW, PATHS), jnp.int32)),
      mesh=_mesh(),
      compiler_params=pltpu.CompilerParams(needs_layout_passes=False),
      scratch_types=[
          pltpu.VMEM((B_PER_W * PATHS,), jnp.float32),
          pltpu.VMEM((NCHUNK, CHUNK), jnp.int32),
          pltpu.VMEM((PATHS,), jnp.int32),
      ],
  )(score)

  dispatched, counts = pl.kernel(
      _dispatch_body,
      out_type=(jax.ShapeDtypeStruct((N_TOKENS, D_MODEL), jnp.float32),
                jax.ShapeDtypeStruct((PATHS,), jnp.int32)),
      mesh=_mesh(),
      compiler_params=pltpu.CompilerParams(needs_layout_passes=False),
      scratch_types=[
          pltpu.VMEM((NW, PATHS), jnp.int32),
          pltpu.VMEM((NCHUNK, CHUNK), jnp.int32),
          pltpu.VMEM((PATHS,), jnp.int32),
          pltpu.VMEM((PATHS,), jnp.int32),
          pltpu.VMEM((CHUNK, D_MODEL), jnp.float32),
          pltpu.VMEM((CHUNK, D_MODEL), jnp.float32),
          pltpu.VMEM((CHUNK, D_MODEL), jnp.float32),
          pltpu.SemaphoreType.DMA,
          pltpu.SemaphoreType.DMA,
          pltpu.SemaphoreType.DMA,
          pltpu.SemaphoreType.DMA,
          pltpu.SemaphoreType.DMA,
          pltpu.SemaphoreType.DMA,
      ],
  )(inputs, route2d, hist)

  return dispatched, route2d.reshape(N_TOKENS), counts


def kernel(inputs):
  n = inputs.shape[0]
  gate_key = jax.random.fold_in(jax.random.key(42), 0)
  score = jax.random.normal(gate_key, (n * PATHS,), dtype=jnp.float32)
  return _run(inputs, score)


# speedup vs baseline: 1.0186x; 1.0186x over previous
import jax
import jax.numpy as jnp
from jax import lax
from jax.experimental import pallas as pl
from jax.experimental.pallas import tpu as pltpu
from jax.experimental.pallas import tpu_sc as plsc

PATHS = 16
N_TOKENS = 16384
D_MODEL = 2048

NC = 2
NS = 16
NW = NC * NS
B_PER_W = N_TOKENS // NW
CHUNK = 16
NCHUNK = B_PER_W // CHUNK
NBUF = 3


def _worker_id():
  return lax.axis_index("s") * NC + lax.axis_index("c")


def _route_body(score_hbm, route_hbm, hist_hbm, score_v, route_v, hist_v):
  wid = _worker_id()
  pltpu.sync_copy(score_hbm.at[pl.ds(wid * B_PER_W * PATHS, B_PER_W * PATHS)],
                  score_v)
  lane = lax.iota(jnp.int32, 16)

  @pl.loop(0, NCHUNK, init_carry=jnp.zeros(16, jnp.int32))
  def hist_loop(g, hist):
    flat = (g * CHUNK + lane) * PATHS
    best = plsc.load_gather(score_v, [flat])
    arg = jnp.zeros(16, jnp.int32)
    for p in range(1, PATHS):
      vals = plsc.load_gather(score_v, [flat + p])
      m = vals > best
      best = jnp.where(m, vals, best)
      arg = jnp.where(m, p, arg)
    route_v[g] = arg
    for p in range(PATHS):
      cnt = plsc.all_reduce_population_count(arg == p)
      hist = hist + jnp.where(lane == p, cnt, 0)
    return hist

  hist_v[...] = hist_loop
  pltpu.sync_copy(route_v, route_hbm.at[pl.ds(wid * NCHUNK, NCHUNK)])
  pltpu.sync_copy(hist_v, hist_hbm.at[wid])


def _dispatch_body(inputs_hbm, route_hbm, hist_hbm,
                   out_hbm, counts_hbm,
                   hist_all_v, route_v, carry_v, tot_v,
                   rows0, rows1, rows2, g0, g1, g2, s0, s1, s2):
  wid = _worker_id()
  row_base = wid * B_PER_W

  bufs = (rows0, rows1, rows2)
  gsems = (g0, g1, g2)
  ssems = (s0, s1, s2)

  def start_in(k, b):
    pltpu.async_copy(
        inputs_hbm.at[pl.ds(row_base + k * CHUNK, CHUNK)], bufs[b], gsems[b])

  for b in range(NBUF):
    start_in(b, b)

  pltpu.sync_copy(hist_hbm, hist_all_v)
  pltpu.sync_copy(route_hbm.at[pl.ds(wid * NCHUNK, NCHUNK)], route_v)
  lane = lax.iota(jnp.int32, 16)

  widv = jnp.full((16,), 0, jnp.int32) + wid
  total = jnp.zeros(16, jnp.int32)
  prior = jnp.zeros(16, jnp.int32)
  for w2 in range(NW):
    h = hist_all_v[w2]
    total = total + h
    prior = prior + jnp.where(jnp.full((16,), w2, jnp.int32) < widv, h, 0)
  path_base = plsc.cumsum(total) - total
  carry_v[...] = path_base + prior

  tot_v[...] = total

  @pl.when(wid == 0)
  def _():
    pltpu.sync_copy(tot_v, counts_hbm)

  @pl.loop(0, NCHUNK, step=NBUF)
  def _(j):
    for b in range(NBUF):
      k = j + b

      @pl.when(k < NCHUNK)
      def _():
        arg = route_v[k]
        dest_base = plsc.load_gather(carry_v, [arg])
        rank = jnp.zeros(16, jnp.int32)
        ghist = jnp.zeros(16, jnp.int32)
        for p in range(PATHS):
          m = arg == p
          cs = plsc.cumsum(m.astype(jnp.int32))
          rank = jnp.where(m, cs - 1, rank)
          ghist = ghist + jnp.where(lane == p,
                                    plsc.all_reduce_population_count(m), 0)
        dest = dest_base + rank
        carry_v[...] = carry_v[...] + ghist
        pltpu.make_async_copy(
            inputs_hbm.at[pl.ds(row_base + k * CHUNK, CHUNK)], bufs[b],
            gsems[b]).wait()
        store = pltpu.async_copy(bufs[b], out_hbm.at[dest], ssems[b])
        store.wait()

        @pl.when(k + NBUF < NCHUNK)
        def _():
          start_in(k + NBUF, b)


def _mesh():
  return plsc.VectorSubcoreMesh(
      core_axis_name="c", subcore_axis_name="s", num_cores=NC,
      num_subcores=NS)


@jax.jit
def _run(inputs, score):
  route2d, hist = pl.kernel(
      _route_body,
      out_type=(jax.ShapeDtypeStruct((N_TOKENS // CHUNK, CHUNK), jnp.int32),
                jax.ShapeDtypeStruct((NW, PATHS), jnp.int32)),
      mesh=_mesh(),
      compiler_params=pltpu.CompilerParams(needs_layout_passes=False),
      scratch_types=[
          pltpu.VMEM((B_PER_W * PATHS,), jnp.float32),
          pltpu.VMEM((NCHUNK, CHUNK), jnp.int32),
          pltpu.VMEM((PATHS,), jnp.int32),
      ],
  )(score)

  dispatched, counts = pl.kernel(
      _dispatch_body,
      out_type=(jax.ShapeDtypeStruct((N_TOKENS, D_MODEL), jnp.float32),
                jax.ShapeDtypeStruct((PATHS,), jnp.int32)),
      mesh=_mesh(),
      compiler_params=pltpu.CompilerParams(needs_layout_passes=False),
      scratch_types=[
          pltpu.VMEM((NW, PATHS), jnp.int32),
          pltpu.VMEM((NCHUNK, CHUNK), jnp.int32),
          pltpu.VMEM((PATHS,), jnp.int32),
          pltpu.VMEM((PATHS,), jnp.int32),
          pltpu.VMEM((CHUNK, D_MODEL), jnp.float32),
          pltpu.VMEM((CHUNK, D_MODEL), jnp.float32),
          pltpu.VMEM((CHUNK, D_MODEL), jnp.float32),
          pltpu.SemaphoreType.DMA,
          pltpu.SemaphoreType.DMA,
          pltpu.SemaphoreType.DMA,
          pltpu.SemaphoreType.DMA,
          pltpu.SemaphoreType.DMA,
          pltpu.SemaphoreType.DMA,
      ],
  )(inputs, route2d, hist)

  return dispatched, route2d.reshape(N_TOKENS), counts


def kernel(inputs):
  n = inputs.shape[0]
  gate_key = jax.random.fold_in(jax.random.key(42), 0)
  score = jax.random.normal(gate_key, (n * PATHS,), dtype=jnp.float32)
  return _run(inputs, score)
